# baseline (speedup 1.0000x reference)
import functools

import jax
import jax.numpy as jnp
from jax.experimental import pallas as pl
from jax.experimental.pallas import tpu as pltpu

_LANES = 128


def _bias_body(f_ref, s_ref, c_ref, o_ref):
    o_ref[...] = (
        jnp.dot(f_ref[...], s_ref[...], preferred_element_type=jnp.float32)
        + c_ref[0, 0]
    )


def kernel(edge_attr, W_edge, b_edge, edge_index, n_nodes, batch):
    E, D = edge_attr.shape
    if E == 0:
        return jnp.zeros((0,), dtype=jnp.float32)

    wv = jnp.mean(W_edge, axis=1)
    c = jnp.mean(b_edge).reshape(1, 1)

    L = _LANES
    G = D * L

    rows = -(-E // L)
    BR = 1000 if rows >= 1000 else rows
    rows_pad = -(-rows // BR) * BR
    E_pad = rows_pad * L
    if E_pad != E:
        edge_attr = jnp.pad(edge_attr, ((0, E_pad - E), (0, 0)))
    F = edge_attr.reshape(rows_pad, G)

    r = jnp.arange(G)
    S = jnp.zeros((G, L), jnp.float32).at[r, r // D].set(wv[r % D])

    out = pl.pallas_call(
        _bias_body,
        grid=(rows_pad // BR,),
        in_specs=[
            pl.BlockSpec((BR, G), lambda i: (i, 0)),
            pl.BlockSpec((G, L), lambda i: (0, 0)),
            pl.BlockSpec(memory_space=pltpu.SMEM),
        ],
        out_specs=pl.BlockSpec((BR, L), lambda i: (i, 0)),
        out_shape=jax.ShapeDtypeStruct((rows_pad, L), jnp.float32),
        compiler_params=pltpu.CompilerParams(
            dimension_semantics=("arbitrary",),
        ),
    )(F, S, c)
    return out.reshape(E_pad)[:E]


# speedup vs baseline: 16.2594x; 16.2594x over previous
import functools

import jax
import jax.numpy as jnp
from jax import lax
from jax.experimental import pallas as pl
from jax.experimental.pallas import tpu as pltpu


def _bias_body(a_ref, w_ref, c_ref, o_ref):
    o_ref[...] = jnp.sum(a_ref[...] * w_ref[...], axis=0) + c_ref[0, 0]


def kernel(edge_attr, W_edge, b_edge, edge_index, n_nodes, batch):
    E, D = edge_attr.shape
    if E == 0:
        return jnp.zeros((0,), dtype=jnp.float32)

    wv = jnp.mean(W_edge, axis=1)
    c = jnp.mean(b_edge).reshape(1, 1)
    wcol = wv[:, None]

    At = edge_attr.T

    BLKL = 128
    for cand in range(131072, 127, -128):
        if E % cand == 0:
            BLKL = cand
            break

    out = pl.pallas_call(
        _bias_body,
        grid=(E // BLKL,),
        in_specs=[
            pl.BlockSpec((D, BLKL), lambda i: (0, i)),
            pl.BlockSpec((D, 1), lambda i: (0, 0)),
            pl.BlockSpec(memory_space=pltpu.SMEM),
        ],
        out_specs=pl.BlockSpec((BLKL,), lambda i: (i,)),
        out_shape=jax.ShapeDtypeStruct((E,), jnp.float32),
        compiler_params=pltpu.CompilerParams(
            dimension_semantics=("arbitrary",),
        ),
    )(At, wcol, c)
    return out
